# baseline (speedup 1.0000x reference)
import functools

import jax
import jax.numpy as jnp
from jax import lax
from jax.experimental import pallas as pl
from jax.experimental.pallas import tpu as pltpu
from jax.experimental.pallas import tpu_sc as plsc

N = 100000
E = 1600000
F_IN = 10
HID = 32
F_OUT = 10

NC, NS = 2, 16
NW = NC * NS
NPAD = 102400
NROW = NPAD // 128
CHUNK = 128
E_PAD = 1605632
ROWS = E_PAD // CHUNK
EPW = E_PAD // NW
CPW = EPW // CHUNK
G = 4
NGROUP = CPW // G
SROWS = 49
NSTAGE = CPW // SROWS
RPT = NPAD // NS

RB = 4096
NBLK = NPAD // RB

_mesh = plsc.VectorSubcoreMesh(core_axis_name="c", subcore_axis_name="s")


def _deg_body(dst_hbm, z_hbm, hists_hbm, hist, pbuf):
    c = lax.axis_index("c")
    s = lax.axis_index("s")
    wid = c * NS + s
    pltpu.sync_copy(z_hbm, hist)
    ones = jnp.ones((16,), jnp.float32)

    def stage_body(j, carry):
        row0 = wid * CPW + j * SROWS
        pltpu.sync_copy(dst_hbm.at[pl.ds(row0, SROWS)], pbuf)

        def inner(r, carry2):
            for k in range(8):
                idx = pbuf[r, pl.ds(k * 16, 16)]
                plsc.addupdate_scatter(hist, [idx], ones)
            return carry2

        return lax.fori_loop(0, SROWS, inner, carry)

    lax.fori_loop(0, NSTAGE, stage_body, 0)
    pltpu.sync_copy(hist, hists_hbm.at[wid])


_deg_call = functools.partial(
    pl.kernel,
    out_type=jax.ShapeDtypeStruct((NW, NPAD), jnp.float32),
    mesh=_mesh,
    compiler_params=pltpu.CompilerParams(
        needs_layout_passes=False, use_tc_tiling_on_sc=False),
    scratch_types=[
        pltpu.VMEM((NPAD,), jnp.float32),
        pltpu.VMEM((SROWS, CHUNK), jnp.int32),
    ],
)(_deg_body)


def _agg_body(src_hbm, dst_hbm, table_hbm, z16_hbm, out_hbm,
              sidx, didx, rows, acc, gsem, ssem):
    c = lax.axis_index("c")
    s = lax.axis_index("s")
    wid = c * NS + s
    pltpu.sync_copy(z16_hbm.at[pl.ds(s * RPT, RPT)],
                    acc.at[pl.ds(s * RPT, RPT)])
    plsc.subcore_barrier()

    base = wid * CPW
    pltpu.sync_copy(src_hbm.at[pl.ds(base, G)], sidx.at[0])
    pltpu.sync_copy(dst_hbm.at[pl.ds(base, G)], didx.at[0])
    for g in range(G):
        pltpu.async_copy(table_hbm.at[sidx.at[0, g]], rows.at[0, g],
                         gsem.at[0])

    def group(grp, carry):
        cur = lax.rem(grp, 2)
        nxt = 1 - cur

        @pl.when(grp > 0)
        def _drain_prev_scatters():
            for g in range(G):
                pltpu.make_async_copy(rows.at[nxt, g],
                                      acc.at[didx.at[nxt, g]], ssem).wait()

        @pl.when(grp < NGROUP - 1)
        def _prefetch():
            row0 = base + (grp + 1) * G
            pltpu.sync_copy(src_hbm.at[pl.ds(row0, G)], sidx.at[nxt])
            pltpu.sync_copy(dst_hbm.at[pl.ds(row0, G)], didx.at[nxt])
            for g in range(G):
                pltpu.async_copy(table_hbm.at[sidx.at[nxt, g]],
                                 rows.at[nxt, g], gsem.at[nxt])

        for g in range(G):
            pltpu.make_async_copy(table_hbm.at[sidx.at[cur, g]],
                                  rows.at[cur, g], gsem.at[cur]).wait()
            pltpu.async_copy(rows.at[cur, g], acc.at[didx.at[cur, g]],
                             ssem, add=True)
        return carry

    lax.fori_loop(0, NGROUP, group, 0)
    last = lax.rem(NGROUP - 1, 2)
    for g in range(G):
        pltpu.make_async_copy(rows.at[last, g],
                              acc.at[didx.at[last, g]], ssem).wait()
    plsc.subcore_barrier()
    pltpu.sync_copy(acc.at[pl.ds(s * RPT, RPT)],
                    out_hbm.at[c, pl.ds(s * RPT, RPT)])


_agg_call = functools.partial(
    pl.kernel,
    out_type=jax.ShapeDtypeStruct((NC, NPAD, 16), jnp.float32),
    mesh=_mesh,
    compiler_params=pltpu.CompilerParams(use_tc_tiling_on_sc=False),
    scratch_types=[
        pltpu.VMEM((2, G, CHUNK), jnp.int32),
        pltpu.VMEM((2, G, CHUNK), jnp.int32),
        pltpu.VMEM((2, G, CHUNK, 16), jnp.float32),
        pltpu.VMEM_SHARED((NPAD, 16), jnp.float32),
        pltpu.SemaphoreType.DMA((2,)),
        pltpu.SemaphoreType.DMA,
    ],
)(_agg_body)


def _expand_body(dinv_hbm, dd16_hbm, dd32a_hbm, dd32b_hbm, dbuf, d16, d32):
    c = lax.axis_index("c")
    s = lax.axis_index("s")
    wid = c * NS + s
    npt = NPAD // NW
    half = npt // 2

    for hf in range(2):
        base = wid * npt + hf * half
        pltpu.sync_copy(dinv_hbm.at[pl.ds(base, half)], dbuf)

        def node16(i, carry):
            for j in range(16):
                k = i * 16 + j
                bb = plsc.load_gather(dbuf, [jnp.full((16,), k, jnp.int32)])
                d16[k] = bb
                r = k // 8
                jj = j % 8
                if jj < 4:
                    d32[0, r, pl.ds(jj * 32, 16)] = bb
                    d32[0, r, pl.ds(jj * 32 + 16, 16)] = bb
                else:
                    d32[1, r, pl.ds((jj - 4) * 32, 16)] = bb
                    d32[1, r, pl.ds((jj - 4) * 32 + 16, 16)] = bb
            return carry

        lax.fori_loop(0, half // 16, node16, 0)
        pltpu.sync_copy(d16, dd16_hbm.at[pl.ds(base, half)])
        pltpu.sync_copy(d32.at[0], dd32a_hbm.at[pl.ds(base // 8, half // 8)])
        pltpu.sync_copy(d32.at[1], dd32b_hbm.at[pl.ds(base // 8, half // 8)])


_expand_call = functools.partial(
    pl.kernel,
    out_type=(jax.ShapeDtypeStruct((NPAD, 16), jnp.float32),
              jax.ShapeDtypeStruct((NROW * 16, 128), jnp.float32),
              jax.ShapeDtypeStruct((NROW * 16, 128), jnp.float32)),
    mesh=_mesh,
    compiler_params=pltpu.CompilerParams(
        needs_layout_passes=False, use_tc_tiling_on_sc=False),
    scratch_types=[
        pltpu.VMEM((NPAD // NW // 2,), jnp.float32),
        pltpu.VMEM((NPAD // NW // 2, 16), jnp.float32),
        pltpu.VMEM((2, NPAD // NW // 16, 128), jnp.float32),
    ],
)(_expand_body)


def _prescale_body(hists_ref, featt_ref, pre1_ref, dinv_ref):
    deg = jnp.sum(hists_ref[...], axis=0, keepdims=True) + 1.0
    dinvr = lax.rsqrt(deg)
    dinv_ref[...] = dinvr
    dcol = dinvr.T
    f = featt_ref[...].T
    pre1_ref[...] = jnp.concatenate(
        [f * dcol, jnp.zeros((RB, 16 - F_IN), jnp.float32)], axis=1)


_prescale_call = pl.pallas_call(
    _prescale_body,
    grid=(NBLK,),
    in_specs=[
        pl.BlockSpec((NW, RB), lambda i: (0, i)),
        pl.BlockSpec((F_IN, RB), lambda i: (0, i)),
    ],
    out_specs=[
        pl.BlockSpec((RB, 16), lambda i: (i, 0)),
        pl.BlockSpec((1, RB), lambda i: (0, i)),
    ],
    out_shape=[
        jax.ShapeDtypeStruct((NPAD, 16), jnp.float32),
        jax.ShapeDtypeStruct((1, NPAD), jnp.float32),
    ],
)


DB = RB // 8


def _mid_body(aggp_ref, pre1_ref, dd32a_ref, dd32b_ref, dd16_ref,
              wb1_ref, b1t_ref, wb2_ref, pre2_ref):
    a = aggp_ref[0] + aggp_ref[1] + pre1_ref[...]
    t = jnp.dot(a, wb1_ref[...], preferred_element_type=jnp.float32)
    b1t = b1t_ref[...]
    ha = jnp.maximum(t[:, :128] * dd32a_ref[...] + b1t[:, :128], 0.0)
    hb = jnp.maximum(t[:, 128:] * dd32b_ref[...] + b1t[:, 128:], 0.0)
    h = jnp.concatenate([ha, hb], axis=1)
    hw = jnp.dot(h, wb2_ref[...], preferred_element_type=jnp.float32)
    pre2_ref[...] = hw * dd16_ref[...]


_mid_call = pl.pallas_call(
    _mid_body,
    grid=(NBLK,),
    in_specs=[
        pl.BlockSpec((NC, DB, 128), lambda i: (0, i, 0)),
        pl.BlockSpec((DB, 128), lambda i: (i, 0)),
        pl.BlockSpec((DB, 128), lambda i: (i, 0)),
        pl.BlockSpec((DB, 128), lambda i: (i, 0)),
        pl.BlockSpec((DB, 128), lambda i: (i, 0)),
        pl.BlockSpec((128, 256), lambda i: (0, 0)),
        pl.BlockSpec((1, 256), lambda i: (0, 0)),
        pl.BlockSpec((256, 128), lambda i: (0, 0)),
    ],
    out_specs=pl.BlockSpec((DB, 128), lambda i: (i, 0)),
    out_shape=jax.ShapeDtypeStruct((NROW * 16, 128), jnp.float32),
)


def _final_body(aggp_ref, pre2_ref, dd16_ref, b2t_ref, out_ref):
    a = aggp_ref[0] + aggp_ref[1] + pre2_ref[...]
    out_ref[...] = a * dd16_ref[...] + b2t_ref[...]


_final_call = pl.pallas_call(
    _final_body,
    grid=(NBLK,),
    in_specs=[
        pl.BlockSpec((NC, DB, 128), lambda i: (0, i, 0)),
        pl.BlockSpec((DB, 128), lambda i: (i, 0)),
        pl.BlockSpec((DB, 128), lambda i: (i, 0)),
        pl.BlockSpec((1, 128), lambda i: (0, 0)),
    ],
    out_specs=pl.BlockSpec((DB, 128), lambda i: (i, 0)),
    out_shape=jax.ShapeDtypeStruct((NROW * 16, 128), jnp.float32),
)


def _outt_body(o_ref, out_ref):
    out_ref[...] = o_ref[...][:, :F_OUT].T


_outt_call = pl.pallas_call(
    _outt_body,
    grid=(NBLK,),
    in_specs=[pl.BlockSpec((RB, 16), lambda i: (i, 0))],
    out_specs=pl.BlockSpec((F_OUT, RB), lambda i: (0, i)),
    out_shape=jax.ShapeDtypeStruct((F_OUT, N), jnp.float32),
)


def kernel(features, edge_index, W1, b1, W2, b2):
    padv = jnp.full((E_PAD - E,), N, jnp.int32)
    dst2d = jnp.concatenate(
        [edge_index[1].astype(jnp.int32), padv]).reshape(ROWS, CHUNK)
    e_b = lax.optimization_barrier(edge_index)
    src2d = jnp.concatenate(
        [e_b[0].astype(jnp.int32), padv]).reshape(ROWS, CHUNK)
    z1 = jnp.zeros((NPAD,), jnp.float32)
    z16 = jnp.zeros((NPAD, 16), jnp.float32)
    feat_t = features.T
    w1p = jnp.pad(W1, ((0, 16 - F_IN), (0, 0)))
    w2p = jnp.pad(W2, ((0, 0), (0, 16 - F_OUT)))
    wb1 = jnp.kron(jnp.eye(8, dtype=jnp.float32), w1p)
    wb2 = jnp.kron(jnp.eye(8, dtype=jnp.float32), w2p)
    b1t = jnp.tile(b1, 8).reshape(1, 256)
    b2t = jnp.tile(jnp.pad(b2, (0, 16 - F_OUT)), 8).reshape(1, 128)

    hists = _deg_call(dst2d, z1)
    pre1, dinv = _prescale_call(hists, feat_t)
    dd16, dd32a, dd32b = _expand_call(dinv.reshape(NPAD))
    dd16d = dd16.reshape(NROW * 16, 128)
    pre1_lin = pre1.reshape(NPAD * 16)
    agg1p = _agg_call(src2d, dst2d, pre1_lin.reshape(NPAD, 16), z16)
    pre2d = _mid_call(agg1p.reshape(NC, NROW * 16, 128),
                      pre1_lin.reshape(NROW * 16, 128),
                      dd32a, dd32b, dd16d, wb1, b1t, wb2)
    agg2p = _agg_call(src2d, dst2d, pre2d.reshape(NPAD, 16), z16)
    outd = _final_call(agg2p.reshape(NC, NROW * 16, 128),
                       pre2d, dd16d, b2t)
    out_t = _outt_call(outd.reshape(NPAD, 16))
    return out_t.T


# speedup vs baseline: 1.0415x; 1.0415x over previous
import functools

import jax
import jax.numpy as jnp
from jax import lax
from jax.experimental import pallas as pl
from jax.experimental.pallas import tpu as pltpu
from jax.experimental.pallas import tpu_sc as plsc

N = 100000
E = 1600000
F_IN = 10
HID = 32
F_OUT = 10

NC, NS = 2, 16
NW = NC * NS
NPAD = 102400
NROW = NPAD // 128
CHUNK = 128
E_PAD = 1605632
ROWS = E_PAD // CHUNK
EPW = E_PAD // NW
CPW = EPW // CHUNK
G = 4
NGROUP = CPW // G
SROWS = 49
NSTAGE = CPW // SROWS
RPT = NPAD // NS

RB = 4096
NBLK = NPAD // RB

_mesh = plsc.VectorSubcoreMesh(core_axis_name="c", subcore_axis_name="s")


def _deg_body(dst_hbm, z_hbm, hists_hbm, hist, pbuf):
    c = lax.axis_index("c")
    s = lax.axis_index("s")
    wid = c * NS + s
    pltpu.sync_copy(z_hbm, hist)
    ones = jnp.ones((16,), jnp.float32)

    def stage_body(j, carry):
        row0 = wid * CPW + j * SROWS
        pltpu.sync_copy(dst_hbm.at[pl.ds(row0, SROWS)], pbuf)

        def inner(r, carry2):
            for k in range(8):
                idx = pbuf[r, pl.ds(k * 16, 16)]
                plsc.addupdate_scatter(hist, [idx], ones)
            return carry2

        return lax.fori_loop(0, SROWS, inner, carry)

    lax.fori_loop(0, NSTAGE, stage_body, 0)
    pltpu.sync_copy(hist, hists_hbm.at[wid])


_deg_call = functools.partial(
    pl.kernel,
    out_type=jax.ShapeDtypeStruct((NW, NPAD), jnp.float32),
    mesh=_mesh,
    compiler_params=pltpu.CompilerParams(
        needs_layout_passes=False, use_tc_tiling_on_sc=False),
    scratch_types=[
        pltpu.VMEM((NPAD,), jnp.float32),
        pltpu.VMEM((SROWS, CHUNK), jnp.int32),
    ],
)(_deg_body)


def _agg_body(src_hbm, dst_hbm, table_hbm, z16_hbm, out_hbm,
              sidx, didx, rows, acc, gsem, ssem):
    c = lax.axis_index("c")
    s = lax.axis_index("s")
    wid = c * NS + s
    pltpu.sync_copy(z16_hbm.at[pl.ds(s * RPT, RPT)],
                    acc.at[pl.ds(s * RPT, RPT)])
    plsc.subcore_barrier()

    base = wid * CPW
    pltpu.sync_copy(src_hbm.at[pl.ds(base, G)], sidx.at[0])
    pltpu.sync_copy(dst_hbm.at[pl.ds(base, G)], didx.at[0])
    for g in range(G):
        pltpu.async_copy(table_hbm.at[sidx.at[0, g]], rows.at[0, g],
                         gsem.at[0])

    def group(grp, carry):
        cur = lax.rem(grp, 2)
        nxt = 1 - cur

        @pl.when(grp > 0)
        def _drain_prev_scatters():
            for g in range(G):
                pltpu.make_async_copy(rows.at[nxt, g],
                                      acc.at[didx.at[nxt, g]], ssem).wait()

        @pl.when(grp < NGROUP - 1)
        def _prefetch():
            row0 = base + (grp + 1) * G
            pltpu.sync_copy(src_hbm.at[pl.ds(row0, G)], sidx.at[nxt])
            pltpu.sync_copy(dst_hbm.at[pl.ds(row0, G)], didx.at[nxt])
            for g in range(G):
                pltpu.async_copy(table_hbm.at[sidx.at[nxt, g]],
                                 rows.at[nxt, g], gsem.at[nxt])

        for g in range(G):
            pltpu.make_async_copy(table_hbm.at[sidx.at[cur, g]],
                                  rows.at[cur, g], gsem.at[cur]).wait()
            pltpu.async_copy(rows.at[cur, g], acc.at[didx.at[cur, g]],
                             ssem, add=True)
        return carry

    lax.fori_loop(0, NGROUP, group, 0)
    last = lax.rem(NGROUP - 1, 2)
    for g in range(G):
        pltpu.make_async_copy(rows.at[last, g],
                              acc.at[didx.at[last, g]], ssem).wait()
    plsc.subcore_barrier()
    pltpu.sync_copy(acc.at[pl.ds(s * RPT, RPT)],
                    out_hbm.at[c, pl.ds(s * RPT, RPT)])


_agg_call = functools.partial(
    pl.kernel,
    out_type=jax.ShapeDtypeStruct((NC, NPAD, 16), jnp.float32),
    mesh=_mesh,
    compiler_params=pltpu.CompilerParams(use_tc_tiling_on_sc=False),
    scratch_types=[
        pltpu.VMEM((2, G, CHUNK), jnp.int32),
        pltpu.VMEM((2, G, CHUNK), jnp.int32),
        pltpu.VMEM((2, G, CHUNK, 16), jnp.float32),
        pltpu.VMEM_SHARED((NPAD, 16), jnp.float32),
        pltpu.SemaphoreType.DMA((2,)),
        pltpu.SemaphoreType.DMA,
    ],
)(_agg_body)


def _expand_body(dinv_hbm, dd16_hbm, dd32a_hbm, dd32b_hbm, dbuf, d16, d32):
    c = lax.axis_index("c")
    s = lax.axis_index("s")
    wid = c * NS + s
    npt = NPAD // NW
    half = npt // 2

    for hf in range(2):
        base = wid * npt + hf * half
        pltpu.sync_copy(dinv_hbm.at[pl.ds(base, half)], dbuf)

        def node16(i, carry):
            for j in range(16):
                k = i * 16 + j
                bb = plsc.load_gather(dbuf, [jnp.full((16,), k, jnp.int32)])
                d16[k] = bb
                r = k // 8
                jj = j % 8
                if jj < 4:
                    d32[0, r, pl.ds(jj * 32, 16)] = bb
                    d32[0, r, pl.ds(jj * 32 + 16, 16)] = bb
                else:
                    d32[1, r, pl.ds((jj - 4) * 32, 16)] = bb
                    d32[1, r, pl.ds((jj - 4) * 32 + 16, 16)] = bb
            return carry

        lax.fori_loop(0, half // 16, node16, 0)
        pltpu.sync_copy(d16, dd16_hbm.at[pl.ds(base, half)])
        pltpu.sync_copy(d32.at[0], dd32a_hbm.at[pl.ds(base // 8, half // 8)])
        pltpu.sync_copy(d32.at[1], dd32b_hbm.at[pl.ds(base // 8, half // 8)])


_expand_call = functools.partial(
    pl.kernel,
    out_type=(jax.ShapeDtypeStruct((NPAD, 16), jnp.float32),
              jax.ShapeDtypeStruct((NROW * 16, 128), jnp.float32),
              jax.ShapeDtypeStruct((NROW * 16, 128), jnp.float32)),
    mesh=_mesh,
    compiler_params=pltpu.CompilerParams(
        needs_layout_passes=False, use_tc_tiling_on_sc=False),
    scratch_types=[
        pltpu.VMEM((NPAD // NW // 2,), jnp.float32),
        pltpu.VMEM((NPAD // NW // 2, 16), jnp.float32),
        pltpu.VMEM((2, NPAD // NW // 16, 128), jnp.float32),
    ],
)(_expand_body)


def _prescale_body(hists_ref, featt_ref, pre1_ref, dinv_ref):
    deg = jnp.sum(hists_ref[...], axis=0, keepdims=True) + 1.0
    dinvr = lax.rsqrt(deg)
    dinv_ref[...] = dinvr
    dcol = dinvr.T
    f = featt_ref[...].T
    pre1_ref[...] = jnp.concatenate(
        [f * dcol, jnp.zeros((RB, 16 - F_IN), jnp.float32)], axis=1)


_prescale_call = pl.pallas_call(
    _prescale_body,
    grid=(NBLK,),
    in_specs=[
        pl.BlockSpec((NW, RB), lambda i: (0, i)),
        pl.BlockSpec((F_IN, RB), lambda i: (0, i)),
    ],
    out_specs=[
        pl.BlockSpec((RB, 16), lambda i: (i, 0)),
        pl.BlockSpec((1, RB), lambda i: (0, i)),
    ],
    out_shape=[
        jax.ShapeDtypeStruct((NPAD, 16), jnp.float32),
        jax.ShapeDtypeStruct((1, NPAD), jnp.float32),
    ],
)


DB = RB // 8


def _mid_body(aggp_ref, pre1_ref, dd32a_ref, dd32b_ref, dd16_ref,
              wb1_ref, b1t_ref, wb2_ref, pre2_ref):
    a = aggp_ref[0] + aggp_ref[1] + pre1_ref[...]
    t = jnp.dot(a, wb1_ref[...], preferred_element_type=jnp.float32)
    b1t = b1t_ref[...]
    ha = jnp.maximum(t[:, :128] * dd32a_ref[...] + b1t[:, :128], 0.0)
    hb = jnp.maximum(t[:, 128:] * dd32b_ref[...] + b1t[:, 128:], 0.0)
    h = jnp.concatenate([ha, hb], axis=1)
    hw = jnp.dot(h, wb2_ref[...], preferred_element_type=jnp.float32)
    pre2_ref[...] = hw * dd16_ref[...]


_mid_call = pl.pallas_call(
    _mid_body,
    grid=(NBLK,),
    in_specs=[
        pl.BlockSpec((NC, DB, 128), lambda i: (0, i, 0)),
        pl.BlockSpec((DB, 128), lambda i: (i, 0)),
        pl.BlockSpec((DB, 128), lambda i: (i, 0)),
        pl.BlockSpec((DB, 128), lambda i: (i, 0)),
        pl.BlockSpec((DB, 128), lambda i: (i, 0)),
        pl.BlockSpec((128, 256), lambda i: (0, 0)),
        pl.BlockSpec((1, 256), lambda i: (0, 0)),
        pl.BlockSpec((256, 128), lambda i: (0, 0)),
    ],
    out_specs=pl.BlockSpec((DB, 128), lambda i: (i, 0)),
    out_shape=jax.ShapeDtypeStruct((NROW * 16, 128), jnp.float32),
)


def _final_body(aggp_ref, pre2_ref, dd16_ref, b2t_ref, out_ref):
    a = aggp_ref[0] + aggp_ref[1] + pre2_ref[...]
    out_ref[...] = a * dd16_ref[...] + b2t_ref[...]


_final_call = pl.pallas_call(
    _final_body,
    grid=(NBLK,),
    in_specs=[
        pl.BlockSpec((NC, DB, 128), lambda i: (0, i, 0)),
        pl.BlockSpec((DB, 128), lambda i: (i, 0)),
        pl.BlockSpec((DB, 128), lambda i: (i, 0)),
        pl.BlockSpec((1, 128), lambda i: (0, 0)),
    ],
    out_specs=pl.BlockSpec((DB, 128), lambda i: (i, 0)),
    out_shape=jax.ShapeDtypeStruct((NROW * 16, 128), jnp.float32),
)


def _outt_body(o_ref, out_ref):
    out_ref[...] = o_ref[...][:, :F_OUT].T


_outt_call = pl.pallas_call(
    _outt_body,
    grid=(NBLK,),
    in_specs=[pl.BlockSpec((RB, 16), lambda i: (i, 0))],
    out_specs=pl.BlockSpec((F_OUT, RB), lambda i: (0, i)),
    out_shape=jax.ShapeDtypeStruct((F_OUT, N), jnp.float32),
)


def kernel(features, edge_index, W1, b1, W2, b2):
    padv = jnp.full((E_PAD - E,), N, jnp.int32)
    e32 = edge_index.astype(jnp.int32)
    src2d = jnp.concatenate([e32[0], padv]).reshape(ROWS, CHUNK)
    dst2d = jnp.concatenate([e32[1], padv]).reshape(ROWS, CHUNK)
    z1 = jnp.zeros((NPAD,), jnp.float32)
    z16 = jnp.zeros((NPAD, 16), jnp.float32)
    feat_t = features.T
    w1p = jnp.pad(W1, ((0, 16 - F_IN), (0, 0)))
    w2p = jnp.pad(W2, ((0, 0), (0, 16 - F_OUT)))
    wb1 = jnp.kron(jnp.eye(8, dtype=jnp.float32), w1p)
    wb2 = jnp.kron(jnp.eye(8, dtype=jnp.float32), w2p)
    b1t = jnp.tile(b1, 8).reshape(1, 256)
    b2t = jnp.tile(jnp.pad(b2, (0, 16 - F_OUT)), 8).reshape(1, 128)

    hists = _deg_call(dst2d, z1)
    pre1, dinv = _prescale_call(hists, feat_t)
    dd16, dd32a, dd32b = _expand_call(dinv.reshape(NPAD))
    dd16d = dd16.reshape(NROW * 16, 128)
    pre1_lin = pre1.reshape(NPAD * 16)
    agg1p = _agg_call(src2d, dst2d, pre1_lin.reshape(NPAD, 16), z16)
    pre2d = _mid_call(agg1p.reshape(NC, NROW * 16, 128),
                      pre1_lin.reshape(NROW * 16, 128),
                      dd32a, dd32b, dd16d, wb1, b1t, wb2)
    agg2p = _agg_call(src2d, dst2d, pre2d.reshape(NPAD, 16), z16)
    outd = _final_call(agg2p.reshape(NC, NROW * 16, 128),
                       pre2d, dd16d, b2t)
    out_t = _outt_call(outd.reshape(NPAD, 16))
    return out_t.T
